# baseline (speedup 1.0000x reference)
import jax
import jax.numpy as jnp
from jax import lax
from jax.experimental import pallas as pl
from jax.experimental.pallas import tpu as pltpu
from jax.experimental.pallas import tpu_sc as plsc

N_NODES = 10000
N_EDGES = 320000
DIM = 128
N_LAYERS = 3

NC = 2
NS = 16
L = 16
HALF = DIM // NC
NP = 10240
TN = NP // NS
NB = 128
EB = 128
G = 8
NG = 20
CHUNKS = G * NG
EPT = CHUNKS * EB
EPAD = NS * EPT
NCH = TN // NB

_F32 = jnp.float32
_I32 = jnp.int32


def _newton_rsqrt(x):
    y = 1.0 / x
    for _ in range(26):
        y = y * (1.5 - 0.5 * x * y * y)
    return y


def _body(src_hbm, dst_hbm, emb_hbm, out_hbm, h_hbm,
          agg, hs_sp, dgo, dgi, ibs, ibd, gbuf, nbuf, obuf, norms,
          onesv, zvec,
          gsem0, gsem1, ssem0, ssem1,
          isems0, isems1, isemd0, isemd1, zsem):
    c = lax.axis_index("c")
    s = lax.axis_index("s")
    nbase = s * TN
    hbase = c * NP + nbase
    coff = c * NP
    z16 = jnp.zeros((L,), _F32)
    gsem = (gsem0, gsem1)
    ssem = (ssem0, ssem1)
    isems = (isems0, isems1)
    isemd = (isemd0, isemd1)

    def _zvec(i, _):
        zvec[pl.ds(i * L, L)] = z16
        return 0
    lax.fori_loop(0, TN // L, _zvec, 0)

    def _ones(i, _):
        onesv[pl.ds(i * L, L)] = jnp.ones((L,), _F32)
        return 0
    lax.fori_loop(0, EB // L, _ones, 0)

    def _zero_gbuf0(i, _):
        for k in range(HALF // L):
            gbuf[0, i, pl.ds(k * L, L)] = z16
        return 0

    pltpu.sync_copy(zvec, dgo.at[pl.ds(nbase, TN)])
    pltpu.sync_copy(zvec, dgi.at[pl.ds(nbase, TN)])
    lax.fori_loop(0, NB, _zero_gbuf0, 0)
    for cb in range(NCH):
        pltpu.sync_copy(gbuf.at[0], agg.at[pl.ds(nbase + cb * NB, NB)])
    plsc.subcore_barrier()

    def _load_idx(g, half):
        pltpu.async_copy(src_hbm.at[s, pl.ds(g * G, G)],
                         ibs.at[half], isems[half])
        pltpu.async_copy(dst_hbm.at[s, pl.ds(g * G, G)],
                         ibd.at[half], isemd[half])

    def _wait_idx(half):
        pltpu.make_async_copy(src_hbm.at[s, pl.ds(0, G)],
                              ibs.at[half], isems[half]).wait()
        pltpu.make_async_copy(dst_hbm.at[s, pl.ds(0, G)],
                              ibd.at[half], isemd[half]).wait()

    def _run_groups(process_group):
        _load_idx(0, 0)
        _load_idx(1, 1)

        def _pair(m, _):
            for half in range(2):
                g = 2 * m + half
                _wait_idx(half)
                process_group(half)
                gn = lax.rem(g + 2, NG)
                _load_idx(gn, half)
            return 0
        lax.fori_loop(0, NG // 2, _pair, 0)
        _wait_idx(0)
        _wait_idx(1)

    def _deg_group(half):
        descs = []
        for jj in range(G):
            descs.append(pltpu.async_copy(
                onesv, dgo.at[ibs.at[half, jj]], gsem[0], add=True))
            descs.append(pltpu.async_copy(
                onesv, dgi.at[ibd.at[half, jj]], gsem[1], add=True))
        for d in descs:
            d.wait()

    with jax.named_scope("deg"):
        _run_groups(_deg_group)
        plsc.subcore_barrier()

    pltpu.sync_copy(dgi.at[pl.ds(nbase, TN)], norms.at[0])
    pltpu.sync_copy(dgo.at[pl.ds(nbase, TN)], norms.at[1])

    def _norm(i, _):
        for d in range(2):
            sl = pl.ds(i * L, L)
            x = jnp.maximum(norms[d, sl], 1.0)
            norms[d, sl] = _newton_rsqrt(x)
        return 0
    lax.fori_loop(0, TN // L, _norm, 0)

    def node_phase(layer):
        if layer > 0:
            lax.fori_loop(0, NB, _zero_gbuf0, 0)
        zdescs = []
        for cb in range(NCH):
            base = nbase + cb * NB
            if layer == 0:
                pltpu.sync_copy(emb_hbm.at[pl.ds(coff + base, NB)], nbuf)
                pltpu.sync_copy(nbuf, out_hbm.at[pl.ds(coff + base, NB)])
            else:
                pltpu.sync_copy(agg.at[pl.ds(base, NB)], nbuf)
                zdescs.append(pltpu.async_copy(
                    gbuf.at[0], agg.at[pl.ds(base, NB)], zsem))
                pltpu.sync_copy(out_hbm.at[pl.ds(coff + base, NB)], obuf)

            def _rows(g, _):
                ndv = norms[0, pl.ds(cb * NB + g * L, L)]
                nsv = norms[1, pl.ds(cb * NB + g * L, L)]
                for t in range(L):
                    i = g * L + t
                    nd = ndv[t]
                    ns = nsv[t]
                    for k in range(HALF // L):
                        sl = pl.ds(k * L, L)
                        v = nbuf[i, sl]
                        if layer == 0:
                            nbuf[i, sl] = v * ns
                        elif layer < N_LAYERS:
                            obuf[i, sl] = obuf[i, sl] + v * nd
                            nbuf[i, sl] = v * (nd * ns)
                        else:
                            w = v * nd
                            obuf[i, sl] = obuf[i, sl] + w
                            nbuf[i, sl] = w
                return 0
            lax.fori_loop(0, NB // L, _rows, 0)

            if layer > 0:
                pltpu.sync_copy(obuf, out_hbm.at[pl.ds(coff + base, NB)])
            if layer < N_LAYERS:
                pltpu.sync_copy(nbuf, hs_sp.at[pl.ds(base, NB)])
            else:
                pltpu.sync_copy(nbuf, h_hbm.at[pl.ds(coff + base, NB)])
        for d in zdescs:
            d.wait()

    def _edge_group(half):
        dg = {}
        sc = {}

        def _scatter(jj):
            b = jj % 2
            dg[jj].wait()
            sc[jj] = pltpu.async_copy(
                gbuf.at[b], agg.at[ibd.at[half, jj]], ssem[b], add=True)

        for jj in range(G):
            b = jj % 2
            if jj >= 2:
                sc[jj - 2].wait()
            dg[jj] = pltpu.async_copy(
                hs_sp.at[ibs.at[half, jj]], gbuf.at[b], gsem[b])
            if jj >= 1:
                _scatter(jj - 1)
        _scatter(G - 1)
        sc[G - 2].wait()
        sc[G - 1].wait()

    with jax.named_scope("node0"):
        node_phase(0)
        plsc.subcore_barrier()
    for layer in range(1, N_LAYERS + 1):
        with jax.named_scope(f"edge{layer}"):
            _run_groups(_edge_group)
            plsc.subcore_barrier()
        with jax.named_scope(f"node{layer}"):
            node_phase(layer)
            if layer < N_LAYERS:
                plsc.subcore_barrier()


_lightgcn_sc = pl.kernel(
    _body,
    out_type=(
        jax.ShapeDtypeStruct((NC * NP, HALF), _F32),
        jax.ShapeDtypeStruct((NC * NP, HALF), _F32),
    ),
    mesh=plsc.VectorSubcoreMesh(core_axis_name="c", subcore_axis_name="s"),
    compiler_params=pltpu.CompilerParams(use_tc_tiling_on_sc=False),
    scratch_types=[
        pltpu.VMEM_SHARED((NP, HALF), _F32),
        pltpu.VMEM_SHARED((NP, HALF), _F32),
        pltpu.VMEM_SHARED((NP,), _F32),
        pltpu.VMEM_SHARED((NP,), _F32),
        pltpu.VMEM((2, G, EB), _I32),
        pltpu.VMEM((2, G, EB), _I32),
        pltpu.VMEM((2, EB, HALF), _F32),
        pltpu.VMEM((NB, HALF), _F32),
        pltpu.VMEM((NB, HALF), _F32),
        pltpu.VMEM((2, TN), _F32),
        pltpu.VMEM((EB,), _F32),
        pltpu.VMEM((TN,), _F32),
    ] + [pltpu.SemaphoreType.DMA] * 9,
)


@jax.jit
def kernel(edge_index, embedding):
    src = edge_index[0].astype(_I32)
    dst = edge_index[1].astype(_I32)
    pad_e = EPAD - N_EDGES
    fill = jnp.full((pad_e,), N_NODES, _I32)
    srcp = jnp.concatenate([src, fill]).reshape(NS, CHUNKS, EB)
    dstp = jnp.concatenate([dst, fill]).reshape(NS, CHUNKS, EB)
    embp = jnp.pad(embedding, ((0, NP - N_NODES), (0, 0)))
    emb_r = embp.reshape(NP, NC, HALF).transpose(1, 0, 2).reshape(NC * NP, HALF)

    out_r, h_r = _lightgcn_sc(srcp, dstp, emb_r)

    def _unsplit(a):
        return (a.reshape(NC, NP, HALF).transpose(1, 0, 2)
                .reshape(NP, DIM)[:N_NODES])

    return (_unsplit(out_r), _unsplit(h_r))
